# baseline (speedup 1.0000x reference)
import jax
import jax.numpy as jnp
from jax import lax
from jax.experimental import pallas as pl
from jax.experimental.pallas import tpu as pltpu
from jax.experimental.pallas import tpu_sc as plsc

EMB = 64
HID = 256
B = 4096
L = 200

NC = 2
NS = 16
NW = NC * NS
BPW = B // NW
NLANE = 16
NGRP = EMB // NLANE
CHUNK = 128
LP = 256
CPR = LP // CHUNK
NCH = BPW * CPR
NBUF = 4


def _sc_body(ids_hbm, table_hbm, out_hbm,
             idx_v, gb0, gb1, gb2, gb3, acc_v, sm0, sm1, sm2, sm3):
    wid = lax.axis_index("s") * NC + lax.axis_index("c")
    pltpu.sync_copy(ids_hbm.at[wid], idx_v)

    gbufs = (gb0, gb1, gb2, gb3)
    sems = (sm0, sm1, sm2, sm3)
    for b in range(NBUF):
        pltpu.async_copy(table_hbm.at[idx_v.at[b]], gbufs[b], sems[b])

    zero = jnp.zeros((NLANE,), jnp.float32)

    def outer(g, carry):
        for half in range(2):
            r = 2 * g + half
            b0, b1 = 2 * half, 2 * half + 1
            for b in (b0, b1):
                j = 4 * g + b
                pltpu.make_async_copy(
                    table_hbm.at[idx_v.at[j]], gbufs[b], sems[b]).wait()

            def body(t, accs):
                lo = tuple(accs[k] + gbufs[b0][t, pl.ds(NLANE * k, NLANE)]
                           for k in range(NGRP))
                hi = tuple(accs[NGRP + k] + gbufs[b1][t, pl.ds(NLANE * k, NLANE)]
                           for k in range(NGRP))
                return lo + hi

            accs = lax.fori_loop(0, CHUNK, body, (zero,) * (2 * NGRP),
                                 unroll=4)
            for k in range(NGRP):
                acc_v[r, pl.ds(NLANE * k, NLANE)] = accs[k] + accs[NGRP + k]

            for b in (b0, b1):
                j = 4 * g + b + NBUF

                @pl.when(j < NCH)
                def _():
                    pltpu.async_copy(
                        table_hbm.at[idx_v.at[j]], gbufs[b], sems[b])
        return carry

    lax.fori_loop(0, NCH // NBUF, outer, 0)
    pltpu.sync_copy(acc_v, out_hbm.at[pl.ds(wid * BPW, BPW)])


_sc_lookup = pl.kernel(
    _sc_body,
    out_type=jax.ShapeDtypeStruct((B, EMB), jnp.float32),
    mesh=plsc.VectorSubcoreMesh(core_axis_name="c", subcore_axis_name="s"),
    compiler_params=pltpu.CompilerParams(use_tc_tiling_on_sc=False),
    scratch_types=[
        pltpu.VMEM((NCH, CHUNK), jnp.int32),
        pltpu.VMEM((CHUNK, EMB), jnp.float32),
        pltpu.VMEM((CHUNK, EMB), jnp.float32),
        pltpu.VMEM((CHUNK, EMB), jnp.float32),
        pltpu.VMEM((CHUNK, EMB), jnp.float32),
        pltpu.VMEM((BPW, EMB), jnp.float32),
        pltpu.SemaphoreType.DMA,
        pltpu.SemaphoreType.DMA,
        pltpu.SemaphoreType.DMA,
        pltpu.SemaphoreType.DMA,
    ],
)


def _mlp_body(x_ref, w1_ref, b1_ref, w2_ref, b2_ref, o_ref):
    x = x_ref[...] * (1.0 / L)
    h = jnp.dot(x, w1_ref[...], preferred_element_type=jnp.float32) + b1_ref[...]
    h = jnp.maximum(h, 0.0)
    y = jnp.dot(h, w2_ref[...], preferred_element_type=jnp.float32) + b2_ref[...]
    o_ref[...] = 1.0 / (1.0 + jnp.exp(-y))


def kernel(input_ids, emb_table, W1, b1, W2, b2):
    ids = jnp.concatenate(
        [input_ids.astype(jnp.int32),
         jnp.zeros((B, LP - L), jnp.int32)], axis=1)
    ids = ids.reshape(NW, NCH, CHUNK)
    sums = _sc_lookup(ids, emb_table)
    return pl.pallas_call(
        _mlp_body,
        out_shape=jax.ShapeDtypeStruct((B, 1), jnp.float32),
    )(sums, W1, b1.reshape(1, HID), W2, b2.reshape(1, 1))


# speedup vs baseline: 6.7493x; 6.7493x over previous
import jax
import jax.numpy as jnp
from jax import lax
from jax.experimental import pallas as pl
from jax.experimental.pallas import tpu as pltpu
from jax.experimental.pallas import tpu_sc as plsc

EMB = 64
HID = 256
B = 4096
L = 200

NC = 2
NS = 16
NW = NC * NS
BPW = B // NW
NLANE = 16
NGRP = EMB // NLANE
CHUNK = 128
LP = 256
CPR = LP // CHUNK
NCH = BPW * CPR
NBUF = 4


def _sc_body(ids_hbm, table_hbm, out_hbm,
             idx_v, gb0, gb1, gb2, gb3, acc_v, sm0, sm1, sm2, sm3):
    wid = lax.axis_index("s") * NC + lax.axis_index("c")
    pltpu.sync_copy(ids_hbm.at[wid], idx_v)

    gbufs = (gb0, gb1, gb2, gb3)
    sems = (sm0, sm1, sm2, sm3)
    for b in range(NBUF):
        pltpu.async_copy(table_hbm.at[idx_v.at[b]], gbufs[b], sems[b])

    zero = jnp.zeros((NLANE,), jnp.float32)

    def outer(g, carry):
        for half in range(2):
            r = 2 * g + half
            b0, b1 = 2 * half, 2 * half + 1
            for b in (b0, b1):
                j = 4 * g + b
                pltpu.make_async_copy(
                    table_hbm.at[idx_v.at[j]], gbufs[b], sems[b]).wait()

            glo, ghi = gbufs[b0], gbufs[b1]

            def body_both(t, accs):
                lo = tuple(accs[k] + glo[t, pl.ds(NLANE * k, NLANE)]
                           for k in range(NGRP))
                hi = tuple(accs[NGRP + k] + ghi[t, pl.ds(NLANE * k, NLANE)]
                           for k in range(NGRP))
                return lo + hi

            def body_first(t, accs):
                return tuple(accs[k] + glo[t, pl.ds(NLANE * k, NLANE)]
                             for k in range(NGRP))

            accs = lax.fori_loop(0, L - CHUNK, body_both,
                                 (zero,) * (2 * NGRP), unroll=4)
            lo = lax.fori_loop(L - CHUNK, CHUNK, body_first,
                               tuple(accs[:NGRP]), unroll=4)
            for k in range(NGRP):
                acc_v[r, pl.ds(NLANE * k, NLANE)] = lo[k] + accs[NGRP + k]

            for b in (b0, b1):
                j = 4 * g + b + NBUF

                @pl.when(j < NCH)
                def _():
                    pltpu.async_copy(
                        table_hbm.at[idx_v.at[j]], gbufs[b], sems[b])
        return carry

    lax.fori_loop(0, NCH // NBUF, outer, 0)
    pltpu.sync_copy(acc_v, out_hbm.at[pl.ds(wid * BPW, BPW)])


_sc_lookup = pl.kernel(
    _sc_body,
    out_type=jax.ShapeDtypeStruct((B, EMB), jnp.float32),
    mesh=plsc.VectorSubcoreMesh(core_axis_name="c", subcore_axis_name="s"),
    compiler_params=pltpu.CompilerParams(use_tc_tiling_on_sc=False),
    scratch_types=[
        pltpu.VMEM((NCH, CHUNK), jnp.int32),
        pltpu.VMEM((CHUNK, EMB), jnp.float32),
        pltpu.VMEM((CHUNK, EMB), jnp.float32),
        pltpu.VMEM((CHUNK, EMB), jnp.float32),
        pltpu.VMEM((CHUNK, EMB), jnp.float32),
        pltpu.VMEM((BPW, EMB), jnp.float32),
        pltpu.SemaphoreType.DMA,
        pltpu.SemaphoreType.DMA,
        pltpu.SemaphoreType.DMA,
        pltpu.SemaphoreType.DMA,
    ],
)


def _mlp_body(x_ref, w1_ref, b1_ref, w2_ref, b2_ref, o_ref):
    x = x_ref[...] * (1.0 / L)
    h = jnp.dot(x, w1_ref[...], preferred_element_type=jnp.float32) + b1_ref[...]
    h = jnp.maximum(h, 0.0)
    y = jnp.dot(h, w2_ref[...], preferred_element_type=jnp.float32) + b2_ref[...]
    o_ref[...] = 1.0 / (1.0 + jnp.exp(-y))


def kernel(input_ids, emb_table, W1, b1, W2, b2):
    ids = input_ids.astype(jnp.int32)
    ids = jnp.concatenate([ids, ids[:, :LP - L]], axis=1)
    ids = ids.reshape(NW, NCH, CHUNK)
    sums = _sc_lookup(ids, emb_table)
    return pl.pallas_call(
        _mlp_body,
        out_shape=jax.ShapeDtypeStruct((B, 1), jnp.float32),
    )(sums, W1, b1.reshape(1, HID), W2, b2.reshape(1, 1))


# speedup vs baseline: 7.3045x; 1.0823x over previous
import jax
import jax.numpy as jnp
from jax import lax
from jax.experimental import pallas as pl
from jax.experimental.pallas import tpu as pltpu
from jax.experimental.pallas import tpu_sc as plsc

EMB = 64
HID = 256
B = 4096
L = 200

NC = 2
NS = 16
NW = NC * NS
BPW = B // NW
NLANE = 16
NGRP = EMB // NLANE
NBUF = 4


def _sc_body(ids_hbm, table_hbm, out_hbm,
             idx_v, gb0, gb1, gb2, gb3, acc_v, sm0, sm1, sm2, sm3):
    wid = lax.axis_index("s") * NC + lax.axis_index("c")
    pltpu.sync_copy(ids_hbm.at[wid], idx_v)

    gbufs = (gb0, gb1, gb2, gb3)
    sems = (sm0, sm1, sm2, sm3)
    for b in range(NBUF):
        pltpu.async_copy(table_hbm.at[idx_v.at[b]], gbufs[b], sems[b])

    zero = jnp.zeros((NLANE,), jnp.float32)

    def outer(g, carry):
        for b in range(NBUF):
            r = 4 * g + b
            pltpu.make_async_copy(
                table_hbm.at[idx_v.at[r]], gbufs[b], sems[b]).wait()

            def red(i, accs):
                i2 = 2 * i
                return (
                    tuple(accs[k]
                          + gbufs[b][i2, pl.ds(NLANE * k, NLANE)]
                          for k in range(NGRP))
                    + tuple(accs[NGRP + k]
                            + gbufs[b][i2 + 1, pl.ds(NLANE * k, NLANE)]
                            for k in range(NGRP)))

            accs = lax.fori_loop(0, L // 2, red, (zero,) * (2 * NGRP),
                                 unroll=4)
            for k in range(NGRP):
                acc_v[r, pl.ds(NLANE * k, NLANE)] = accs[k] + accs[NGRP + k]

            rn = r + NBUF

            @pl.when(rn < BPW)
            def _():
                pltpu.async_copy(
                    table_hbm.at[idx_v.at[rn]], gbufs[b], sems[b])
        return carry

    lax.fori_loop(0, BPW // NBUF, outer, 0)
    pltpu.sync_copy(acc_v, out_hbm.at[pl.ds(wid * BPW, BPW)])


_sc_lookup = pl.kernel(
    _sc_body,
    out_type=jax.ShapeDtypeStruct((B, EMB), jnp.float32),
    mesh=plsc.VectorSubcoreMesh(core_axis_name="c", subcore_axis_name="s"),
    compiler_params=pltpu.CompilerParams(use_tc_tiling_on_sc=False),
    scratch_types=[
        pltpu.VMEM((BPW, L), jnp.int32),
        pltpu.VMEM((L, EMB), jnp.float32),
        pltpu.VMEM((L, EMB), jnp.float32),
        pltpu.VMEM((L, EMB), jnp.float32),
        pltpu.VMEM((L, EMB), jnp.float32),
        pltpu.VMEM((BPW, EMB), jnp.float32),
        pltpu.SemaphoreType.DMA,
        pltpu.SemaphoreType.DMA,
        pltpu.SemaphoreType.DMA,
        pltpu.SemaphoreType.DMA,
    ],
)


def _mlp_body(x_ref, w1_ref, b1_ref, w2_ref, b2_ref, o_ref):
    x = x_ref[...] * (1.0 / L)
    h = jnp.dot(x, w1_ref[...], preferred_element_type=jnp.float32) + b1_ref[...]
    h = jnp.maximum(h, 0.0)
    y = jnp.dot(h, w2_ref[...], preferred_element_type=jnp.float32) + b2_ref[...]
    o_ref[...] = 1.0 / (1.0 + jnp.exp(-y))


def kernel(input_ids, emb_table, W1, b1, W2, b2):
    ids = input_ids.astype(jnp.int32).reshape(NW, BPW, L)
    sums = _sc_lookup(ids, emb_table)
    return pl.pallas_call(
        _mlp_body,
        out_shape=jax.ShapeDtypeStruct((B, 1), jnp.float32),
    )(sums, W1, b1.reshape(1, HID), W2, b2.reshape(1, 1))
